# initial kernel scaffold (unmeasured)
import jax
import jax.numpy as jnp
from jax import lax
from jax.experimental import pallas as pl
from jax.experimental.pallas import tpu as pltpu


def kernel(
    x,
):
    def body(*refs):
        pass

    out_shape = jax.ShapeDtypeStruct(..., jnp.float32)
    return pl.pallas_call(body, out_shape=out_shape)(...)


# baseline (device time: 209052 ns/iter reference)
import jax
import jax.numpy as jnp
from jax import lax
from jax.experimental import pallas as pl
from jax.experimental.pallas import tpu as pltpu


def kernel(x):
    m, n = x.shape

    def body(x_ref, out_ref, comm_ref, comm2_ref, send_sems, recv_sems):
        my_x = lax.axis_index("x")
        my_y = lax.axis_index("y")
        x_partner = (1 - my_x, my_y)
        y_partner = (my_x, 1 - my_y)

        barrier_sem = pltpu.get_barrier_semaphore()
        for nbr in (x_partner, y_partner):
            pl.semaphore_signal(
                barrier_sem, inc=1,
                device_id=nbr, device_id_type=pl.DeviceIdType.MESH,
            )
        pl.semaphore_wait(barrier_sem, 2)

        rdma1 = pltpu.make_async_remote_copy(
            src_ref=x_ref,
            dst_ref=comm_ref,
            send_sem=send_sems.at[0],
            recv_sem=recv_sems.at[0],
            device_id=x_partner,
            device_id_type=pl.DeviceIdType.MESH,
        )
        rdma1.start()
        rdma1.wait()

        comm_ref[...] = x_ref[...] + comm_ref[...]

        rdma2 = pltpu.make_async_remote_copy(
            src_ref=comm_ref,
            dst_ref=comm2_ref,
            send_sem=send_sems.at[1],
            recv_sem=recv_sems.at[1],
            device_id=y_partner,
            device_id_type=pl.DeviceIdType.MESH,
        )
        rdma2.start()

        @pl.when(my_y == 0)
        def _():
            out_ref[:, 0:n] = comm_ref[...]

        @pl.when(my_y == 1)
        def _():
            out_ref[:, n : 2 * n] = comm_ref[...]

        rdma2.wait()

        @pl.when(my_y == 0)
        def _():
            out_ref[:, n : 2 * n] = comm2_ref[...]

        @pl.when(my_y == 1)
        def _():
            out_ref[:, 0:n] = comm2_ref[...]

    return pl.pallas_call(
        body,
        out_shape=jax.ShapeDtypeStruct((m, 2 * n), x.dtype),
        in_specs=[pl.BlockSpec(memory_space=pltpu.VMEM)],
        out_specs=pl.BlockSpec(memory_space=pltpu.VMEM),
        scratch_shapes=[
            pltpu.VMEM((m, n), x.dtype),
            pltpu.VMEM((m, n), x.dtype),
            pltpu.SemaphoreType.DMA((2,)),
            pltpu.SemaphoreType.DMA((2,)),
        ],
        compiler_params=pltpu.CompilerParams(
            collective_id=0,
            vmem_limit_bytes=100 * 1024 * 1024,
        ),
    )(x)


# device time: 123925 ns/iter; 1.6869x vs baseline; 1.6869x over previous
import jax
import jax.numpy as jnp
from jax import lax
from jax.experimental import pallas as pl
from jax.experimental.pallas import tpu as pltpu

N_CHUNKS = 16


def kernel(x):
    m, n = x.shape
    mc = m // N_CHUNKS

    def body(x_ref, out_ref, comm_ref, send1, recv1, send2, recv2):
        my_x = lax.axis_index("x")
        my_y = lax.axis_index("y")
        x_partner = (1 - my_x, my_y)
        y_partner = (my_x, 1 - my_y)

        barrier_sem = pltpu.get_barrier_semaphore()
        for nbr in (x_partner, y_partner):
            pl.semaphore_signal(
                barrier_sem, inc=1,
                device_id=nbr, device_id_type=pl.DeviceIdType.MESH,
            )
        pl.semaphore_wait(barrier_sem, 2)

        rdma1 = []
        for c in range(N_CHUNKS):
            rows = pl.ds(c * mc, mc)
            r = pltpu.make_async_remote_copy(
                src_ref=x_ref.at[rows],
                dst_ref=comm_ref.at[rows],
                send_sem=send1.at[c],
                recv_sem=recv1.at[c],
                device_id=x_partner,
                device_id_type=pl.DeviceIdType.MESH,
            )
            r.start()
            rdma1.append(r)

        rdma2 = []
        for c in range(N_CHUNKS):
            rows = pl.ds(c * mc, mc)
            rdma1[c].wait_recv()
            comm_ref[rows] = x_ref[rows] + comm_ref[rows]

            d0 = pltpu.make_async_remote_copy(
                src_ref=comm_ref.at[rows],
                dst_ref=out_ref.at[rows, pl.ds(0, n)],
                send_sem=send2.at[c],
                recv_sem=recv2.at[c],
                device_id=y_partner,
                device_id_type=pl.DeviceIdType.MESH,
            )
            d1 = pltpu.make_async_remote_copy(
                src_ref=comm_ref.at[rows],
                dst_ref=out_ref.at[rows, pl.ds(n, n)],
                send_sem=send2.at[c],
                recv_sem=recv2.at[c],
                device_id=y_partner,
                device_id_type=pl.DeviceIdType.MESH,
            )

            @pl.when(my_y == 0)
            def _(d0=d0, rows=rows):
                d0.start()
                out_ref[rows, pl.ds(0, n)] = comm_ref[rows]

            @pl.when(my_y == 1)
            def _(d1=d1, rows=rows):
                d1.start()
                out_ref[rows, pl.ds(n, n)] = comm_ref[rows]

            rdma2.append((d0, d1))

        for c in range(N_CHUNKS):
            d0, d1 = rdma2[c]

            @pl.when(my_y == 0)
            def _(d0=d0, d1=d1):
                d0.wait_send()
                d1.wait_recv()

            @pl.when(my_y == 1)
            def _(d0=d0, d1=d1):
                d1.wait_send()
                d0.wait_recv()

        for c in range(N_CHUNKS):
            rdma1[c].wait_send()

    return pl.pallas_call(
        body,
        out_shape=jax.ShapeDtypeStruct((m, 2 * n), x.dtype),
        in_specs=[pl.BlockSpec(memory_space=pltpu.VMEM)],
        out_specs=pl.BlockSpec(memory_space=pltpu.VMEM),
        scratch_shapes=[
            pltpu.VMEM((m, n), x.dtype),
            pltpu.SemaphoreType.DMA((N_CHUNKS,)),
            pltpu.SemaphoreType.DMA((N_CHUNKS,)),
            pltpu.SemaphoreType.DMA((N_CHUNKS,)),
            pltpu.SemaphoreType.DMA((N_CHUNKS,)),
        ],
        compiler_params=pltpu.CompilerParams(
            collective_id=0,
            vmem_limit_bytes=100 * 1024 * 1024,
        ),
    )(x)
